# baseline (speedup 1.0000x reference)
import functools

import numpy as np
import jax
import jax.numpy as jnp
from jax import lax
from jax.experimental import pallas as pl
from jax.experimental.pallas import tpu as pltpu
from jax.experimental.pallas import tpu_sc as plsc

EMBED = 64
SEQ = 200
SEQ_PAD = 256
LANES = 16
NUM_WORKERS = 32
ROW_F32 = SEQ * EMBED
OUT_MINOR = ROW_F32 // 8
CHUNKS = ((0, 104), (104, 96))


def _positional_encoding_np(length, depth):
    half = depth / 2
    positions = np.arange(length)[:, np.newaxis]
    depths = np.arange(half)[np.newaxis, :] / half
    angle_rates = 1 / 10000 ** depths
    angle_rads = positions * angle_rates
    return np.concatenate(
        [np.sin(angle_rads), np.cos(angle_rads)], axis=-1
    ).astype(np.float32)


def _make_sc_kernel(batch):
    assert batch % NUM_WORKERS == 0
    b_per_w = batch // NUM_WORKERS
    scale = float(EMBED) ** 0.5

    @functools.partial(
        pl.kernel,
        mesh=plsc.VectorSubcoreMesh(core_axis_name="c", subcore_axis_name="s"),
        out_type=jax.ShapeDtypeStruct((batch, 8, OUT_MINOR), jnp.float32),
        scratch_types=[
            pltpu.VMEM((b_per_w, SEQ_PAD), jnp.int32),
            pltpu.VMEM((b_per_w * SEQ_PAD,), jnp.int32),
            pltpu.VMEM((SEQ, EMBED), jnp.float32),
            pltpu.VMEM((SEQ, EMBED), jnp.float32),
            pltpu.VMEM((8, OUT_MINOR), jnp.float32),
            pltpu.VMEM((8, OUT_MINOR), jnp.float32),
            pltpu.SemaphoreType.DMA,
            pltpu.SemaphoreType.DMA,
            pltpu.SemaphoreType.DMA,
            pltpu.SemaphoreType.DMA,
        ],
        compiler_params=pltpu.CompilerParams(
            use_tc_tiling_on_sc=False, needs_layout_passes=False
        ),
    )
    def sc_kernel(x_hbm, table_hbm, out_hbm,
                  xf_v, idx_v, rows0, rows1, out0, out1,
                  gsem0, gsem1, osem0, osem1):
        wid = lax.axis_index("s") * 2 + lax.axis_index("c")
        base = wid * b_per_w
        pltpu.sync_copy(x_hbm.at[pl.ds(base, b_per_w)], xf_v)

        def conv_row(i, c2):
            ib = i * SEQ_PAD
            for c in range(SEQ_PAD // LANES):
                idx_v[pl.ds(ib + c * LANES, LANES)] = xf_v[
                    i, pl.ds(c * LANES, LANES)
                ]
            return c2

        lax.fori_loop(0, b_per_w, conv_row, 0)

        rows = (rows0, rows1)
        outs = (out0, out1)
        gsems = (gsem0, gsem1)
        osems = (osem0, osem1)

        def start_gather(i, slot):
            for off, n in CHUNKS:
                pltpu.make_async_copy(
                    table_hbm.at[idx_v.at[pl.ds(i * SEQ_PAD + off, n)]],
                    rows[slot].at[pl.ds(off, n)],
                    gsems[slot],
                ).start()

        def wait_gather(slot):
            for off, n in CHUNKS:
                pltpu.make_async_copy(
                    table_hbm.at[idx_v.at[pl.ds(off, n)]],
                    rows[slot].at[pl.ds(off, n)],
                    gsems[slot],
                ).wait()

        def wait_out(i, slot):
            pltpu.make_async_copy(
                outs[slot], out_hbm.at[base + i], osems[slot]
            ).wait()

        start_gather(0, 0)
        start_gather(1, 1)

        C = EMBED // LANES

        def per_pair(i2, carry):
            for slot in range(2):
                i = i2 * 2 + slot
                wait_gather(slot)

                @pl.when(i2 > 0)
                def _():
                    wait_out(i, slot)

                rv, ov = rows[slot], outs[slot]

                def per_block(rr, c2):
                    r0 = rr * 25

                    def per_rowpair(qq, c3):
                        r = r0 + qq * 2
                        fb = qq * 2 * EMBED
                        vals = [
                            rv[r + dr, pl.ds(c * LANES, LANES)]
                            for dr in range(2)
                            for c in range(C)
                        ]
                        res = [v * scale for v in vals]
                        for k in range(2 * C):
                            ov[rr, pl.ds(fb + k * LANES, LANES)] = res[k]
                        return c3

                    lax.fori_loop(0, 12, per_rowpair, 0)
                    r = r0 + 24
                    fb = 24 * EMBED
                    vals = [rv[r, pl.ds(c * LANES, LANES)] for c in range(C)]
                    res = [v * scale for v in vals]
                    for c in range(C):
                        ov[rr, pl.ds(fb + c * LANES, LANES)] = res[c]
                    return c2

                lax.fori_loop(0, 8, per_block, 0)

                pltpu.make_async_copy(
                    ov, out_hbm.at[base + i], osems[slot]
                ).start()

                @pl.when(i2 < (b_per_w // 2) - 1)
                def _():
                    start_gather(i + 2, slot)
            return carry

        lax.fori_loop(0, b_per_w // 2, per_pair, 0)
        wait_out(b_per_w - 2, 0)
        wait_out(b_per_w - 1, 1)

    return sc_kernel


def kernel(x, table):
    batch, seq = x.shape
    assert seq == SEQ and table.shape[1] == EMBED
    pe = jnp.asarray(_positional_encoding_np(SEQ, EMBED))
    x_pad = jnp.pad(x.astype(jnp.int32), ((0, 0), (0, SEQ_PAD - SEQ)))
    out = _make_sc_kernel(batch)(x_pad, table)
    return out.reshape(batch, SEQ, EMBED) + pe[None, :, :]


# speedup vs baseline: 1.0347x; 1.0347x over previous
import functools

import numpy as np
import jax
import jax.numpy as jnp
from jax import lax
from jax.experimental import pallas as pl
from jax.experimental.pallas import tpu as pltpu
from jax.experimental.pallas import tpu_sc as plsc

EMBED = 64
SEQ = 200
SEQ_PAD = 256
LANES = 16
NUM_WORKERS = 32
ROW_F32 = SEQ * EMBED
OUT_MINOR = ROW_F32 // 8
CHUNKS = ((0, 104), (104, 96))


def _positional_encoding_np(length, depth):
    half = depth / 2
    positions = np.arange(length)[:, np.newaxis]
    depths = np.arange(half)[np.newaxis, :] / half
    angle_rates = 1 / 10000 ** depths
    angle_rads = positions * angle_rates
    return np.concatenate(
        [np.sin(angle_rads), np.cos(angle_rads)], axis=-1
    ).astype(np.float32)


def _make_sc_kernel(batch):
    assert batch % NUM_WORKERS == 0
    b_per_w = batch // NUM_WORKERS
    scale = float(EMBED) ** 0.5

    @functools.partial(
        pl.kernel,
        mesh=plsc.VectorSubcoreMesh(core_axis_name="c", subcore_axis_name="s"),
        out_type=jax.ShapeDtypeStruct((batch, 8, OUT_MINOR), jnp.float32),
        scratch_types=[
            pltpu.VMEM((b_per_w, SEQ_PAD), jnp.int32),
            pltpu.VMEM((b_per_w * SEQ_PAD,), jnp.int32),
            pltpu.VMEM((SEQ, EMBED), jnp.float32),
            pltpu.VMEM((SEQ, EMBED), jnp.float32),
            pltpu.VMEM((8, OUT_MINOR), jnp.float32),
            pltpu.VMEM((8, OUT_MINOR), jnp.float32),
            pltpu.VMEM((8, OUT_MINOR), jnp.float32),
            pltpu.SemaphoreType.DMA,
            pltpu.SemaphoreType.DMA,
            pltpu.SemaphoreType.DMA,
            pltpu.SemaphoreType.DMA,
        ],
        compiler_params=pltpu.CompilerParams(
            use_tc_tiling_on_sc=False, needs_layout_passes=False
        ),
    )
    def sc_kernel(x_hbm, table_hbm, pe_hbm, out_hbm,
                  xf_v, idx_v, rows0, rows1, out0, out1, pe_v,
                  gsem0, gsem1, osem0, osem1):
        wid = lax.axis_index("s") * 2 + lax.axis_index("c")
        base = wid * b_per_w
        pltpu.sync_copy(pe_hbm, pe_v)
        pltpu.sync_copy(x_hbm.at[pl.ds(base, b_per_w)], xf_v)

        def conv_row(i, c2):
            ib = i * SEQ_PAD
            for c in range(SEQ_PAD // LANES):
                idx_v[pl.ds(ib + c * LANES, LANES)] = xf_v[
                    i, pl.ds(c * LANES, LANES)
                ]
            return c2

        lax.fori_loop(0, b_per_w, conv_row, 0)

        rows = (rows0, rows1)
        outs = (out0, out1)
        gsems = (gsem0, gsem1)
        osems = (osem0, osem1)

        def start_gather(i, slot):
            for off, n in CHUNKS:
                pltpu.make_async_copy(
                    table_hbm.at[idx_v.at[pl.ds(i * SEQ_PAD + off, n)]],
                    rows[slot].at[pl.ds(off, n)],
                    gsems[slot],
                ).start()

        def wait_gather(slot):
            for off, n in CHUNKS:
                pltpu.make_async_copy(
                    table_hbm.at[idx_v.at[pl.ds(off, n)]],
                    rows[slot].at[pl.ds(off, n)],
                    gsems[slot],
                ).wait()

        def wait_out(i, slot):
            pltpu.make_async_copy(
                outs[slot], out_hbm.at[base + i], osems[slot]
            ).wait()

        start_gather(0, 0)
        start_gather(1, 1)

        C = EMBED // LANES

        def per_pair(i2, carry):
            for slot in range(2):
                i = i2 * 2 + slot
                wait_gather(slot)

                @pl.when(i2 > 0)
                def _():
                    wait_out(i, slot)

                rv, ov = rows[slot], outs[slot]

                def per_block(rr, c2):
                    r0 = rr * 25

                    def per_rowpair(qq, c3):
                        r = r0 + qq * 2
                        fb = qq * 2 * EMBED
                        vals = [
                            rv[r + dr, pl.ds(c * LANES, LANES)]
                            for dr in range(2)
                            for c in range(C)
                        ]
                        pes = [
                            pe_v[rr, pl.ds(fb + k * LANES, LANES)]
                            for k in range(2 * C)
                        ]
                        res = [v * scale + p for v, p in zip(vals, pes)]
                        for k in range(2 * C):
                            ov[rr, pl.ds(fb + k * LANES, LANES)] = res[k]
                        return c3

                    lax.fori_loop(0, 12, per_rowpair, 0)
                    r = r0 + 24
                    fb = 24 * EMBED
                    vals = [rv[r, pl.ds(c * LANES, LANES)] for c in range(C)]
                    pes = [
                        pe_v[rr, pl.ds(fb + c * LANES, LANES)] for c in range(C)
                    ]
                    res = [v * scale + p for v, p in zip(vals, pes)]
                    for c in range(C):
                        ov[rr, pl.ds(fb + c * LANES, LANES)] = res[c]
                    return c2

                lax.fori_loop(0, 8, per_block, 0)

                pltpu.make_async_copy(
                    ov, out_hbm.at[base + i], osems[slot]
                ).start()

                @pl.when(i2 < (b_per_w // 2) - 1)
                def _():
                    start_gather(i + 2, slot)
            return carry

        lax.fori_loop(0, b_per_w // 2, per_pair, 0)
        wait_out(b_per_w - 2, 0)
        wait_out(b_per_w - 1, 1)

    return sc_kernel


def kernel(x, table):
    batch, seq = x.shape
    assert seq == SEQ and table.shape[1] == EMBED
    pe = jnp.asarray(_positional_encoding_np(SEQ, EMBED).reshape(8, OUT_MINOR))
    x_pad = jnp.pad(x.astype(jnp.int32), ((0, 0), (0, SEQ_PAD - SEQ)))
    out = _make_sc_kernel(batch)(x_pad, table, pe)
    return out.reshape(batch, SEQ, EMBED)


# speedup vs baseline: 1.0446x; 1.0095x over previous
import functools

import numpy as np
import jax
import jax.numpy as jnp
from jax import lax
from jax.experimental import pallas as pl
from jax.experimental.pallas import tpu as pltpu
from jax.experimental.pallas import tpu_sc as plsc

EMBED = 64
SEQ = 200
SEQ_PAD = 256
LANES = 16
NUM_WORKERS = 32
ROW_F32 = SEQ * EMBED
OUT_MINOR = ROW_F32 // 8
CHUNKS = ((0, 104), (104, 96))


def _positional_encoding_np(length, depth):
    half = depth / 2
    positions = np.arange(length)[:, np.newaxis]
    depths = np.arange(half)[np.newaxis, :] / half
    angle_rates = 1 / 10000 ** depths
    angle_rads = positions * angle_rates
    return np.concatenate(
        [np.sin(angle_rads), np.cos(angle_rads)], axis=-1
    ).astype(np.float32)


def _make_sc_kernel(batch):
    assert batch % NUM_WORKERS == 0
    b_per_w = batch // NUM_WORKERS
    scale = float(EMBED) ** 0.5

    @functools.partial(
        pl.kernel,
        mesh=plsc.VectorSubcoreMesh(core_axis_name="c", subcore_axis_name="s"),
        out_type=jax.ShapeDtypeStruct((batch, ROW_F32), jnp.float32),
        scratch_types=[
            pltpu.VMEM((b_per_w, SEQ_PAD), jnp.int32),
            pltpu.VMEM((b_per_w * SEQ_PAD,), jnp.int32),
            pltpu.VMEM((SEQ, EMBED), jnp.float32),
            pltpu.VMEM((SEQ, EMBED), jnp.float32),
            pltpu.VMEM((ROW_F32,), jnp.float32),
            pltpu.VMEM((ROW_F32,), jnp.float32),
            pltpu.VMEM((ROW_F32,), jnp.float32),
            pltpu.SemaphoreType.DMA,
            pltpu.SemaphoreType.DMA,
            pltpu.SemaphoreType.DMA,
            pltpu.SemaphoreType.DMA,
        ],
        compiler_params=pltpu.CompilerParams(
            use_tc_tiling_on_sc=False, needs_layout_passes=False
        ),
    )
    def sc_kernel(x_hbm, table_hbm, pe_hbm, out_hbm,
                  xf_v, idx_v, rows0, rows1, out0, out1, pe_v,
                  gsem0, gsem1, osem0, osem1):
        wid = lax.axis_index("s") * 2 + lax.axis_index("c")
        base = wid * b_per_w
        pltpu.sync_copy(pe_hbm, pe_v)
        pltpu.sync_copy(x_hbm.at[pl.ds(base, b_per_w)], xf_v)

        def conv_row(i, c2):
            ib = i * SEQ_PAD
            for c in range(SEQ_PAD // LANES):
                idx_v[pl.ds(ib + c * LANES, LANES)] = xf_v[
                    i, pl.ds(c * LANES, LANES)
                ]
            return c2

        lax.fori_loop(0, b_per_w, conv_row, 0)

        rows = (rows0, rows1)
        outs = (out0, out1)
        gsems = (gsem0, gsem1)
        osems = (osem0, osem1)

        def start_gather(i, slot):
            for off, n in CHUNKS:
                pltpu.make_async_copy(
                    table_hbm.at[idx_v.at[pl.ds(i * SEQ_PAD + off, n)]],
                    rows[slot].at[pl.ds(off, n)],
                    gsems[slot],
                ).start()

        def wait_gather(slot):
            for off, n in CHUNKS:
                pltpu.make_async_copy(
                    table_hbm.at[idx_v.at[pl.ds(off, n)]],
                    rows[slot].at[pl.ds(off, n)],
                    gsems[slot],
                ).wait()

        def wait_out(i, slot):
            pltpu.make_async_copy(
                outs[slot], out_hbm.at[base + i], osems[slot]
            ).wait()

        start_gather(0, 0)
        start_gather(1, 1)

        C = EMBED // LANES

        def per_pair(i2, carry):
            for slot in range(2):
                i = i2 * 2 + slot
                wait_gather(slot)

                @pl.when(i2 > 0)
                def _():
                    wait_out(i, slot)

                rv, ov = rows[slot], outs[slot]

                def per_rowpair(qq, c3):
                    r = qq * 2
                    fb = r * EMBED
                    vals = [
                        rv[r + dr, pl.ds(c * LANES, LANES)]
                        for dr in range(2)
                        for c in range(C)
                    ]
                    pes = [
                        pe_v[pl.ds(fb + k * LANES, LANES)]
                        for k in range(2 * C)
                    ]
                    res = [v * scale + p for v, p in zip(vals, pes)]
                    for k in range(2 * C):
                        ov[pl.ds(fb + k * LANES, LANES)] = res[k]
                    return c3

                lax.fori_loop(0, SEQ // 2, per_rowpair, 0)

                pltpu.make_async_copy(
                    ov, out_hbm.at[base + i], osems[slot]
                ).start()

                @pl.when(i2 < (b_per_w // 2) - 1)
                def _():
                    start_gather(i + 2, slot)
            return carry

        lax.fori_loop(0, b_per_w // 2, per_pair, 0)
        wait_out(b_per_w - 2, 0)
        wait_out(b_per_w - 1, 1)

    return sc_kernel


def kernel(x, table):
    batch, seq = x.shape
    assert seq == SEQ and table.shape[1] == EMBED
    pe = jnp.asarray(_positional_encoding_np(SEQ, EMBED).reshape(-1))
    x_pad = jnp.pad(x.astype(jnp.int32), ((0, 0), (0, SEQ_PAD - SEQ)))
    out = _make_sc_kernel(batch)(x_pad, table, pe)
    return out.reshape(batch, SEQ, EMBED)


# speedup vs baseline: 1.0484x; 1.0037x over previous
import functools

import numpy as np
import jax
import jax.numpy as jnp
from jax import lax
from jax.experimental import pallas as pl
from jax.experimental.pallas import tpu as pltpu
from jax.experimental.pallas import tpu_sc as plsc

EMBED = 64
SEQ = 200
SEQ_PAD = 256
LANES = 16
NUM_WORKERS = 32
ROW_F32 = SEQ * EMBED
CHUNKS = ((0, 104), (104, 96))


def _positional_encoding_np(length, depth):
    half = depth / 2
    positions = np.arange(length)[:, np.newaxis]
    depths = np.arange(half)[np.newaxis, :] / half
    angle_rates = 1 / 10000 ** depths
    angle_rads = positions * angle_rates
    return np.concatenate(
        [np.sin(angle_rads), np.cos(angle_rads)], axis=-1
    ).astype(np.float32)


def _make_sc_kernel(batch):
    assert batch % NUM_WORKERS == 0
    b_per_w = batch // NUM_WORKERS
    scale = float(EMBED) ** 0.5

    @functools.partial(
        pl.kernel,
        mesh=plsc.VectorSubcoreMesh(core_axis_name="c", subcore_axis_name="s"),
        out_type=jax.ShapeDtypeStruct((batch, ROW_F32), jnp.float32),
        scratch_types=[
            pltpu.VMEM((b_per_w, SEQ_PAD), jnp.int32),
            pltpu.VMEM((b_per_w * SEQ_PAD,), jnp.int32),
            pltpu.VMEM((SEQ, EMBED), jnp.float32),
            pltpu.VMEM((SEQ, EMBED), jnp.float32),
            pltpu.VMEM((ROW_F32,), jnp.float32),
            pltpu.VMEM((ROW_F32,), jnp.float32),
            pltpu.VMEM((ROW_F32,), jnp.float32),
            pltpu.SemaphoreType.DMA,
            pltpu.SemaphoreType.DMA,
            pltpu.SemaphoreType.DMA,
            pltpu.SemaphoreType.DMA,
        ],
        compiler_params=pltpu.CompilerParams(
            use_tc_tiling_on_sc=False, needs_layout_passes=False
        ),
    )
    def sc_kernel(x_hbm, table_hbm, pe_hbm, out_hbm,
                  xf_v, idx_v, rows0, rows1, out0, out1, pe_v,
                  gsem0, gsem1, osem0, osem1):
        wid = lax.axis_index("s") * 2 + lax.axis_index("c")
        base = wid * b_per_w
        pltpu.sync_copy(pe_hbm, pe_v)
        pltpu.sync_copy(x_hbm.at[pl.ds(base, b_per_w)], xf_v)

        def conv_row(i, c2):
            ib = i * SEQ_PAD
            for c in range(SEQ_PAD // LANES):
                idx_v[pl.ds(ib + c * LANES, LANES)] = xf_v[
                    i, pl.ds(c * LANES, LANES)
                ]
            return c2

        lax.fori_loop(0, b_per_w, conv_row, 0)

        rows = (rows0, rows1)
        outs = (out0, out1)
        gsems = (gsem0, gsem1)
        osems = (osem0, osem1)

        def start_gather(i, slot):
            for off, n in CHUNKS:
                pltpu.make_async_copy(
                    table_hbm.at[idx_v.at[pl.ds(i * SEQ_PAD + off, n)]],
                    rows[slot].at[pl.ds(off, n)],
                    gsems[slot],
                ).start()

        def wait_gather(slot):
            for off, n in CHUNKS:
                pltpu.make_async_copy(
                    table_hbm.at[idx_v.at[pl.ds(off, n)]],
                    rows[slot].at[pl.ds(off, n)],
                    gsems[slot],
                ).wait()

        def wait_out(i, slot):
            pltpu.make_async_copy(
                outs[slot], out_hbm.at[base + i], osems[slot]
            ).wait()

        start_gather(0, 0)
        start_gather(1, 1)

        C = EMBED // LANES

        def per_pair(i2, carry):
            for slot in range(2):
                i = i2 * 2 + slot
                wait_gather(slot)

                @pl.when(i2 > 0)
                def _():
                    wait_out(i, slot)

                rv, ov = rows[slot], outs[slot]

                def per_rowpair(qq, c3):
                    r = qq * 2
                    fb = r * EMBED
                    vals = [
                        rv[r + dr, pl.ds(c * LANES, LANES)]
                        for dr in range(2)
                        for c in range(C)
                    ]
                    pes = [
                        pe_v[pl.ds(fb + k * LANES, LANES)]
                        for k in range(2 * C)
                    ]
                    res = [v * scale + p for v, p in zip(vals, pes)]
                    for k in range(2 * C):
                        ov[pl.ds(fb + k * LANES, LANES)] = res[k]
                    return c3

                lax.fori_loop(0, SEQ // 2, per_rowpair, 0)

                pltpu.make_async_copy(
                    ov, out_hbm.at[base + i], osems[slot]
                ).start()

                @pl.when(i2 < (b_per_w // 2) - 1)
                def _():
                    start_gather(i + 2, slot)
            return carry

        lax.fori_loop(0, b_per_w // 2, per_pair, 0)
        wait_out(b_per_w - 2, 0)
        wait_out(b_per_w - 1, 1)

    return sc_kernel


def kernel(x, table):
    batch, seq = x.shape
    assert seq == SEQ and table.shape[1] == EMBED
    pe = jnp.asarray(_positional_encoding_np(SEQ, EMBED).reshape(-1))
    x_pad = jnp.pad(x.astype(jnp.int32), ((0, 0), (0, SEQ_PAD - SEQ)))
    out = _make_sc_kernel(batch)(x_pad, table, pe)
    return out.reshape(batch, SEQ, EMBED)
